# baseline (speedup 1.0000x reference)
import jax
import jax.numpy as jnp
from jax.experimental import pallas as pl
from jax.experimental.pallas import tpu as pltpu
from jax.experimental.pallas import tpu_sc as plsc
from jax.experimental.layout import Format, Layout, with_layout_constraint

BATCH = 16384
HIST = 200
EMB = 32
N = BATCH * HIST
WINDOW = 128
K = 8


def _sc_gather(table_hbm, idx_hbm, out_hbm, sem):
    def body(i_vmem, o_vmem):
        copies = [
            pltpu.async_copy(
                table_hbm.at[i_vmem.at[j]],
                o_vmem.at[pl.ds(j * WINDOW, WINDOW)],
                sem,
            )
            for j in range(K)
        ]
        for c in copies:
            c.wait()

    pltpu.emit_pipeline(
        body,
        grid=(N // (WINDOW * K),),
        in_specs=[pl.BlockSpec((K, WINDOW), index_map=lambda i: (i, 0))],
        out_specs=[pl.BlockSpec((K * WINDOW, EMB), index_map=lambda i: (i, 0))],
        core_axis_name=("c", "s"),
        dimension_semantics=(pltpu.PARALLEL,),
    )(idx_hbm, out_hbm)


def _impl(prev_action, action_emb_weight):
    idx = prev_action.reshape(N // WINDOW, WINDOW).astype(jnp.int32)
    mesh = plsc.VectorSubcoreMesh(core_axis_name="c", subcore_axis_name="s")
    out = pl.kernel(
        _sc_gather,
        out_type=jax.ShapeDtypeStruct((N, EMB), jnp.float32),
        mesh=mesh,
        scratch_types=[pltpu.SemaphoreType.DMA],
        compiler_params=pltpu.CompilerParams(use_tc_tiling_on_sc=False),
    )(action_emb_weight, idx)
    return out.reshape(BATCH, HIST, EMB)


_plain_jit = jax.jit(_impl)
_pinned_jit_cache = {}


def kernel(prev_action, action_emb_weight):
    try:
        dev = next(iter(prev_action.devices()))
    except Exception:
        return _plain_jit(prev_action, action_emb_weight)
    if dev not in _pinned_jit_cache:
        fmt = Format(
            Layout(major_to_minor=(0, 1, 2), tiling=((16,),)),
            jax.sharding.SingleDeviceSharding(dev),
        )
        _pinned_jit_cache[dev] = jax.jit(_impl, out_shardings=fmt)
    return _pinned_jit_cache[dev](prev_action, action_emb_weight)


# speedup vs baseline: 1.9827x; 1.9827x over previous
import jax
import jax.numpy as jnp
from jax.experimental import pallas as pl
from jax.experimental.pallas import tpu as pltpu
from jax.experimental.pallas import tpu_sc as plsc

BATCH = 16384
HIST = 200
EMB = 32
N = BATCH * HIST
WINDOW = 128
K = 8
BB = 128


def _sc_gather(table_hbm, idx_hbm, out_hbm, sem):
    def body(i_vmem, o_vmem):
        copies = [
            pltpu.async_copy(
                table_hbm.at[i_vmem.at[j]],
                o_vmem.at[pl.ds(j * WINDOW, WINDOW)],
                sem,
            )
            for j in range(K)
        ]
        for c in copies:
            c.wait()

    pltpu.emit_pipeline(
        body,
        grid=(N // (WINDOW * K),),
        in_specs=[pl.BlockSpec((K, WINDOW), index_map=lambda i: (i, 0))],
        out_specs=[pl.BlockSpec((K * WINDOW, EMB), index_map=lambda i: (i, 0))],
        core_axis_name=("c", "s"),
        dimension_semantics=(pltpu.PARALLEL,),
    )(idx_hbm, out_hbm)


def _tc_transpose(x_ref, o_ref):
    x = x_ref[...].reshape(BB, HIST * EMB // 128, 128)
    t = jnp.transpose(x, (1, 2, 0))
    o_ref[...] = t.reshape(HIST, EMB, BB)


def _impl(prev_action, action_emb_weight):
    idx = prev_action.reshape(N // WINDOW, WINDOW).astype(jnp.int32)
    mesh = plsc.VectorSubcoreMesh(core_axis_name="c", subcore_axis_name="s")
    interm = pl.kernel(
        _sc_gather,
        out_type=jax.ShapeDtypeStruct((N, EMB), jnp.float32),
        mesh=mesh,
        scratch_types=[pltpu.SemaphoreType.DMA],
        compiler_params=pltpu.CompilerParams(use_tc_tiling_on_sc=False),
    )(action_emb_weight, idx)

    x2 = interm.reshape(N * EMB // 128, 128)
    t = pl.pallas_call(
        _tc_transpose,
        out_shape=jax.ShapeDtypeStruct((HIST, EMB, BATCH), jnp.float32),
        grid=(BATCH // BB,),
        in_specs=[
            pl.BlockSpec((BB * HIST * EMB // 128, 128), lambda i: (i, 0))
        ],
        out_specs=pl.BlockSpec((HIST, EMB, BB), lambda i: (0, 0, i)),
    )(x2)
    return jnp.transpose(t, (2, 0, 1))


kernel = jax.jit(_impl)
